# Initial kernel scaffold; baseline (speedup 1.0000x reference)
import jax
import jax.numpy as jnp
from jax.experimental import pallas as pl


def kernel(X, edge_index, edge_weight):
    raise NotImplementedError("write your pallas kernel here")


# speedup vs baseline: 3.5946x; 3.5946x over previous
import functools

import jax
import jax.numpy as jnp
from jax import lax
from jax.experimental import pallas as pl
from jax.experimental.pallas import tpu as pltpu
from jax.experimental.pallas import tpu_sc as plsc

N = 10000
D = 128
LAYERS = 2
NC = 2
NS = 16
CHUNK = 128
NP_ = 10240
ROWS_PER_TILE = NP_ // NS
WSTRIPE = 128


def _sc_agg(ego, col, row, w):
    e_pad = col.shape[0]
    per_tile = e_pad // (NC * NS)
    n_chunks = per_tile // CHUNK
    mesh = plsc.VectorSubcoreMesh(core_axis_name="c", subcore_axis_name="s")

    @functools.partial(
        pl.kernel,
        mesh=mesh,
        out_type=jax.ShapeDtypeStruct((NC * NP_, D), jnp.float32),
        scratch_types=[
            pltpu.VMEM_SHARED((NP_, D), jnp.float32),
            pltpu.VMEM((CHUNK,), jnp.int32),
            pltpu.VMEM((CHUNK,), jnp.int32),
            pltpu.VMEM((CHUNK, D), jnp.float32),
            pltpu.VMEM((CHUNK,), jnp.float32),
            pltpu.SemaphoreType.DMA,
        ],
    )
    def k(ego_hbm, col_hbm, row_hbm, w_hbm, out_hbm,
          agg_sh, colv, rowv, rows, wv, sem):
        c = lax.axis_index("c")
        s = lax.axis_index("s")
        rbase = s * ROWS_PER_TILE

        def zrow(i, carry):
            for d8 in range(D // 16):
                rows[i, pl.ds(d8 * 16, 16)] = jnp.zeros((16,), jnp.float32)
            return carry
        lax.fori_loop(0, WSTRIPE, zrow, 0)
        for kk in range(ROWS_PER_TILE // WSTRIPE):
            pltpu.sync_copy(rows.at[pl.ds(0, WSTRIPE)],
                            agg_sh.at[pl.ds(rbase + kk * WSTRIPE, WSTRIPE)])
        plsc.subcore_barrier()

        ebase = (c * NS + s) * per_tile

        def body(j, carry):
            off = ebase + j * CHUNK
            pltpu.sync_copy(col_hbm.at[pl.ds(off, CHUNK)], colv)
            pltpu.sync_copy(w_hbm.at[pl.ds(off, CHUNK)], wv)
            pltpu.async_copy(ego_hbm.at[colv], rows, sem).wait()

            def mgrp(k, inner):
                wvec = wv[pl.ds(k * 16, 16)]
                for jj in range(16):
                    i = k * 16 + jj
                    lane = jnp.full((16,), jj, dtype=jnp.int32)
                    wsplat = wvec.at[lane].get(mode="promise_in_bounds")
                    for d8 in range(D // 16):
                        sl = pl.ds(d8 * 16, 16)
                        rows[i, sl] = rows[i, sl] * wsplat
                return inner
            lax.fori_loop(0, CHUNK // 16, mgrp, 0)

            pltpu.sync_copy(row_hbm.at[pl.ds(off, CHUNK)], rowv)
            pltpu.sync_copy(rows, agg_sh.at[rowv], add=True)
            return carry
        lax.fori_loop(0, n_chunks, body, 0)
        plsc.subcore_barrier()

        for kk in range(ROWS_PER_TILE // WSTRIPE):
            r0 = rbase + kk * WSTRIPE
            pltpu.sync_copy(agg_sh.at[pl.ds(r0, WSTRIPE)],
                            rows.at[pl.ds(0, WSTRIPE)])
            pltpu.sync_copy(rows.at[pl.ds(0, WSTRIPE)],
                            out_hbm.at[pl.ds(c * NP_ + r0, WSTRIPE)])

    return k(ego, col, row, w)


def _tc_update(partials, ego, acc, scale):
    bn = 1000

    def body(p_ref, e_ref, a_ref, eo_ref, ao_ref):
        agg = p_ref[0] + p_ref[1]
        e_new = agg + agg * e_ref[...]
        eo_ref[...] = e_new
        ao_ref[...] = (a_ref[...] + e_new) * scale

    return pl.pallas_call(
        body,
        grid=(N // bn,),
        in_specs=[
            pl.BlockSpec((2, bn, D), lambda i: (0, i, 0)),
            pl.BlockSpec((bn, D), lambda i: (i, 0)),
            pl.BlockSpec((bn, D), lambda i: (i, 0)),
        ],
        out_specs=[
            pl.BlockSpec((bn, D), lambda i: (i, 0)),
            pl.BlockSpec((bn, D), lambda i: (i, 0)),
        ],
        out_shape=[
            jax.ShapeDtypeStruct((N, D), jnp.float32),
            jax.ShapeDtypeStruct((N, D), jnp.float32),
        ],
    )(partials, ego, acc)


def kernel(X, edge_index, edge_weight):
    row = edge_index[0]
    col = edge_index[1]
    e = row.shape[0]
    gran = NC * NS * CHUNK
    e_pad = ((e + gran - 1) // gran) * gran
    pad = e_pad - e
    if pad:
        row = jnp.concatenate([row, jnp.zeros((pad,), jnp.int32)])
        col = jnp.concatenate([col, jnp.zeros((pad,), jnp.int32)])
        w = jnp.concatenate([edge_weight, jnp.zeros((pad,), jnp.float32)])
    else:
        w = edge_weight

    ego = X
    acc = X
    for layer in range(LAYERS):
        partials = _sc_agg(ego, col, row, w).reshape(2, NP_, D)[:, :N]
        scale = (1.0 / (LAYERS + 1)) if layer == LAYERS - 1 else 1.0
        ego, acc = _tc_update(partials, ego, acc, scale)
    return acc


# speedup vs baseline: 10.6481x; 2.9622x over previous
import functools

import jax
import jax.numpy as jnp
from jax import lax
from jax.experimental import pallas as pl
from jax.experimental.pallas import tpu as pltpu
from jax.experimental.pallas import tpu_sc as plsc

N = 10000
D = 128
LAYERS = 2
NC = 2
NS = 16
CHUNK = 128
NP_ = 10240
ROWS_PER_TILE = NP_ // NS
WSTRIPE = 128
CPT = 80


def _sc_agg(ego, col, row, w):
    mesh = plsc.VectorSubcoreMesh(core_axis_name="c", subcore_axis_name="s")

    @functools.partial(
        pl.kernel,
        mesh=mesh,
        out_type=jax.ShapeDtypeStruct((NC * NP_, D), jnp.float32),
        scratch_types=[
            pltpu.VMEM_SHARED((NP_, D), jnp.float32),
            pltpu.VMEM((CPT, CHUNK), jnp.int32),
            pltpu.VMEM((2, CHUNK), jnp.int32),
            pltpu.VMEM((2, CHUNK), jnp.float32),
            pltpu.VMEM((CHUNK, D), jnp.float32),
            pltpu.VMEM((CHUNK, D), jnp.float32),
            pltpu.SemaphoreType.DMA,
            pltpu.SemaphoreType.DMA,
            pltpu.SemaphoreType.DMA,
            pltpu.SemaphoreType.DMA,
        ],
    )
    def k(ego_hbm, col_hbm, row_hbm, w_hbm, out_hbm,
          agg_sh, colv, rowr, wr, rows0, rows1, sem0, sem1, semi0, semi1):
        c = lax.axis_index("c")
        s = lax.axis_index("s")
        rbase = s * ROWS_PER_TILE
        tid = c * NS + s
        cb = tid * CPT

        pltpu.sync_copy(col_hbm.at[pl.ds(cb, CPT)], colv)
        for par in range(2):
            pltpu.sync_copy(row_hbm.at[cb + par], rowr.at[par])
            pltpu.sync_copy(w_hbm.at[cb + par], wr.at[par])

        def zrow(i, carry):
            for d8 in range(D // 16):
                rows0[i, pl.ds(d8 * 16, 16)] = jnp.zeros((16,), jnp.float32)
            return carry
        lax.fori_loop(0, WSTRIPE, zrow, 0)
        for kk in range(ROWS_PER_TILE // WSTRIPE):
            pltpu.sync_copy(rows0.at[pl.ds(0, WSTRIPE)],
                            agg_sh.at[pl.ds(rbase + kk * WSTRIPE, WSTRIPE)])

        pltpu.make_async_copy(ego_hbm.at[colv.at[0]], rows0, sem0).start()
        pltpu.make_async_copy(ego_hbm.at[colv.at[1]], rows1, sem1).start()
        plsc.subcore_barrier()

        def process(q, par, rows, sem, semi, ring_wait):
            pltpu.make_async_copy(ego_hbm.at[colv.at[q]], rows, sem).wait()
            if ring_wait:
                pltpu.make_async_copy(row_hbm.at[cb + q], rowr.at[par], semi).wait()
                pltpu.make_async_copy(w_hbm.at[cb + q], wr.at[par], semi).wait()

            def mgrp(kk, inner):
                wvec = wr[par, pl.ds(kk * 16, 16)]
                for jj in range(16):
                    i = kk * 16 + jj
                    lane = jnp.full((16,), jj, dtype=jnp.int32)
                    wsplat = wvec.at[lane].get(mode="promise_in_bounds")
                    for d8 in range(D // 16):
                        sl = pl.ds(d8 * 16, 16)
                        rows[i, sl] = rows[i, sl] * wsplat
                return inner
            lax.fori_loop(0, CHUNK // 16, mgrp, 0)
            pltpu.sync_copy(rows, agg_sh.at[rowr.at[par]], add=True)

        def refill(q, par, semi):
            pltpu.make_async_copy(row_hbm.at[cb + q], rowr.at[par], semi).start()
            pltpu.make_async_copy(w_hbm.at[cb + q], wr.at[par], semi).start()

        def body(p, ring_wait):
            q0 = 2 * p
            process(q0, 0, rows0, sem0, semi0, ring_wait)
            refill(q0 + 2, 0, semi0)
            pltpu.make_async_copy(ego_hbm.at[colv.at[q0 + 2]], rows0, sem0).start()
            process(q0 + 1, 1, rows1, sem1, semi1, ring_wait)
            refill(q0 + 3, 1, semi1)
            pltpu.make_async_copy(ego_hbm.at[colv.at[q0 + 3]], rows1, sem1).start()

        body(0, False)

        def bodyn(p, carry):
            body(p, True)
            return carry
        lax.fori_loop(1, CPT // 2 - 1, bodyn, 0)
        process(CPT - 2, 0, rows0, sem0, semi0, True)
        process(CPT - 1, 1, rows1, sem1, semi1, True)
        plsc.subcore_barrier()

        for kk in range(ROWS_PER_TILE // WSTRIPE):
            r0 = rbase + kk * WSTRIPE
            pltpu.sync_copy(agg_sh.at[pl.ds(r0, WSTRIPE)],
                            rows0.at[pl.ds(0, WSTRIPE)])
            pltpu.sync_copy(rows0.at[pl.ds(0, WSTRIPE)],
                            out_hbm.at[pl.ds(c * NP_ + r0, WSTRIPE)])

    return k(ego, col, row, w)


def _tc_update(partials, ego, acc, scale):
    bn = 1000

    def body(p_ref, e_ref, a_ref, eo_ref, ao_ref):
        agg = p_ref[0] + p_ref[1]
        e_new = agg + agg * e_ref[...]
        eo_ref[...] = e_new
        ao_ref[...] = (a_ref[...] + e_new) * scale

    return pl.pallas_call(
        body,
        grid=(N // bn,),
        in_specs=[
            pl.BlockSpec((2, bn, D), lambda i: (0, i, 0)),
            pl.BlockSpec((bn, D), lambda i: (i, 0)),
            pl.BlockSpec((bn, D), lambda i: (i, 0)),
        ],
        out_specs=[
            pl.BlockSpec((bn, D), lambda i: (i, 0)),
            pl.BlockSpec((bn, D), lambda i: (i, 0)),
        ],
        out_shape=[
            jax.ShapeDtypeStruct((N, D), jnp.float32),
            jax.ShapeDtypeStruct((N, D), jnp.float32),
        ],
    )(partials, ego, acc)


def kernel(X, edge_index, edge_weight):
    row = edge_index[0]
    col = edge_index[1]
    e = row.shape[0]
    e_pad = NC * NS * CPT * CHUNK
    pad = e_pad - e
    if pad > 0:
        pad_idx = jnp.arange(pad, dtype=jnp.int32) % N
        row = jnp.concatenate([row, pad_idx])
        col = jnp.concatenate([col, pad_idx])
        w = jnp.concatenate([edge_weight, jnp.zeros((pad,), jnp.float32)])
    else:
        w = edge_weight
    row = row.reshape(-1, CHUNK)
    col = col.reshape(-1, CHUNK)
    w = w.reshape(-1, CHUNK)

    ego = X
    acc = X
    for layer in range(LAYERS):
        partials = _sc_agg(ego, col, row, w).reshape(2, NP_, D)[:, :N]
        scale = (1.0 / (LAYERS + 1)) if layer == LAYERS - 1 else 1.0
        ego, acc = _tc_update(partials, ego, acc, scale)
    return acc


# speedup vs baseline: 11.1484x; 1.0470x over previous
import functools

import jax
import jax.numpy as jnp
from jax import lax
from jax.experimental import pallas as pl
from jax.experimental.pallas import tpu as pltpu
from jax.experimental.pallas import tpu_sc as plsc

N = 10000
D = 128
LAYERS = 2
NC = 2
NS = 16
CHUNK = 128
NP_ = 10240
ROWS_PER_TILE = NP_ // NS
WSTRIPE = 128
CPT = 80


def _sc_agg(ego, col, row, w):
    mesh = plsc.VectorSubcoreMesh(core_axis_name="c", subcore_axis_name="s")

    @functools.partial(
        pl.kernel,
        mesh=mesh,
        out_type=jax.ShapeDtypeStruct((NC * NP_, D), jnp.float32),
        scratch_types=[
            pltpu.VMEM_SHARED((NP_, D), jnp.float32),
            pltpu.VMEM((CPT, CHUNK), jnp.int32),
            pltpu.VMEM((2, CHUNK), jnp.int32),
            pltpu.VMEM((2, CHUNK), jnp.float32),
            pltpu.VMEM((CHUNK, D), jnp.float32),
            pltpu.VMEM((CHUNK, D), jnp.float32),
            pltpu.SemaphoreType.DMA,
            pltpu.SemaphoreType.DMA,
            pltpu.SemaphoreType.DMA,
            pltpu.SemaphoreType.DMA,
        ],
    )
    def k(ego_hbm, col_hbm, row_hbm, w_hbm, out_hbm,
          agg_sh, colv, rowr, wr, rows0, rows1, sem0, sem1, semi0, semi1):
        c = lax.axis_index("c")
        s = lax.axis_index("s")
        rbase = s * ROWS_PER_TILE
        tid = c * NS + s
        cb = tid * CPT

        pltpu.sync_copy(col_hbm.at[pl.ds(cb, CPT)], colv)
        for par in range(2):
            pltpu.sync_copy(row_hbm.at[cb + par], rowr.at[par])
            pltpu.sync_copy(w_hbm.at[cb + par], wr.at[par])

        def zrow(i, carry):
            for d8 in range(D // 16):
                rows0[i, pl.ds(d8 * 16, 16)] = jnp.zeros((16,), jnp.float32)
            return carry
        lax.fori_loop(0, WSTRIPE, zrow, 0)
        for kk in range(ROWS_PER_TILE // WSTRIPE):
            pltpu.sync_copy(rows0.at[pl.ds(0, WSTRIPE)],
                            agg_sh.at[pl.ds(rbase + kk * WSTRIPE, WSTRIPE)])

        pltpu.make_async_copy(ego_hbm.at[colv.at[0]], rows0, sem0).start()
        pltpu.make_async_copy(ego_hbm.at[colv.at[1]], rows1, sem1).start()
        plsc.subcore_barrier()

        def process(q, par, rows, sem, semi, ring_wait):
            pltpu.make_async_copy(ego_hbm.at[colv.at[q]], rows, sem).wait()
            if ring_wait:
                pltpu.make_async_copy(row_hbm.at[cb + q], rowr.at[par], semi).wait()
                pltpu.make_async_copy(w_hbm.at[cb + q], wr.at[par], semi).wait()

            def mgrp(kk, inner):
                wvec = wr[par, pl.ds(kk * 16, 16)]
                for jj in range(16):
                    i = kk * 16 + jj
                    lane = jnp.full((16,), jj, dtype=jnp.int32)
                    wsplat = wvec.at[lane].get(mode="promise_in_bounds")
                    for d8 in range(D // 16):
                        sl = pl.ds(d8 * 16, 16)
                        rows[i, sl] = rows[i, sl] * wsplat
                return inner
            lax.fori_loop(0, CHUNK // 16, mgrp, 0)
            pltpu.sync_copy(rows, agg_sh.at[rowr.at[par]], add=True)

        def refill(q, par, semi):
            pltpu.make_async_copy(row_hbm.at[cb + q], rowr.at[par], semi).start()
            pltpu.make_async_copy(w_hbm.at[cb + q], wr.at[par], semi).start()

        def body(p, ring_wait):
            q0 = 2 * p
            process(q0, 0, rows0, sem0, semi0, ring_wait)
            refill(q0 + 2, 0, semi0)
            pltpu.make_async_copy(ego_hbm.at[colv.at[q0 + 2]], rows0, sem0).start()
            process(q0 + 1, 1, rows1, sem1, semi1, ring_wait)
            refill(q0 + 3, 1, semi1)
            pltpu.make_async_copy(ego_hbm.at[colv.at[q0 + 3]], rows1, sem1).start()

        body(0, False)

        def bodyn(p, carry):
            body(p, True)
            return carry
        lax.fori_loop(1, CPT // 2 - 1, bodyn, 0)
        process(CPT - 2, 0, rows0, sem0, semi0, True)
        process(CPT - 1, 1, rows1, sem1, semi1, True)
        plsc.subcore_barrier()

        for kk in range(ROWS_PER_TILE // WSTRIPE):
            r0 = rbase + kk * WSTRIPE
            pltpu.sync_copy(agg_sh.at[pl.ds(r0, WSTRIPE)],
                            rows0.at[pl.ds(0, WSTRIPE)])
            pltpu.sync_copy(rows0.at[pl.ds(0, WSTRIPE)],
                            out_hbm.at[pl.ds(c * NP_ + r0, WSTRIPE)])

    return k(ego, col, row, w)


def _tc_update(partials, ego, acc, scale):
    bn = 1000

    def body(p_ref, e_ref, a_ref, eo_ref, ao_ref):
        agg = p_ref[0] + p_ref[1]
        e_new = agg + agg * e_ref[...]
        eo_ref[...] = e_new
        ao_ref[...] = (a_ref[...] + e_new) * scale

    return pl.pallas_call(
        body,
        grid=(N // bn,),
        in_specs=[
            pl.BlockSpec((2, bn, D), lambda i: (0, i, 0)),
            pl.BlockSpec((bn, D), lambda i: (i, 0)),
            pl.BlockSpec((bn, D), lambda i: (i, 0)),
        ],
        out_specs=[
            pl.BlockSpec((bn, D), lambda i: (i, 0)),
            pl.BlockSpec((bn, D), lambda i: (i, 0)),
        ],
        out_shape=[
            jax.ShapeDtypeStruct((N, D), jnp.float32),
            jax.ShapeDtypeStruct((N, D), jnp.float32),
        ],
    )(partials, ego, acc)


def kernel(X, edge_index, edge_weight):
    row = edge_index[0]
    col = edge_index[1]
    e = row.shape[0]
    e_pad = NC * NS * CPT * CHUNK
    pad = e_pad - e
    if pad > 0:
        pad_idx = jnp.arange(pad, dtype=jnp.int32) % N
        row = jnp.concatenate([row, pad_idx])
        col = jnp.concatenate([col, pad_idx])
        w = jnp.concatenate([edge_weight, jnp.zeros((pad,), jnp.float32)])
    else:
        w = edge_weight
    row = row.reshape(-1, CHUNK)
    col = col.reshape(-1, CHUNK)
    w = w.reshape(-1, CHUNK)

    ego = X
    acc = X
    for layer in range(LAYERS):
        partials = _sc_agg(ego, col, row, w).reshape(2, NP_, D)
        scale = (1.0 / (LAYERS + 1)) if layer == LAYERS - 1 else 1.0
        ego, acc = _tc_update(partials, ego, acc, scale)
    return acc
